# Initial kernel scaffold; baseline (speedup 1.0000x reference)
import jax
import jax.numpy as jnp
from jax.experimental import pallas as pl


def kernel(x, edge_index, W, A):
    raise NotImplementedError("write your pallas kernel here")


# speedup vs baseline: 7.2026x; 7.2026x over previous
import functools

import jax
import jax.numpy as jnp
from jax import lax
from jax.experimental import pallas as pl
from jax.experimental.pallas import tpu as pltpu
from jax.experimental.pallas import tpu_sc as plsc

N = 10000
E = 320000
D = 128

NC = 2
NS = 16
L = 16
NW = NC * NS

N_PAD = 10240
HALF = N_PAD // NC
ACC_ROWS = 5248
ACC_PER_TILE = ACC_ROWS // NS
JUNK = HALF + 32
NR = N_PAD // D
CH = 128
CPT = 2
E_PAD = 327680
EPT = E_PAD // NS
NTRIP = EPT // (CH * CPT)
PAD_RECV = 10232
DC_ROWS = 2 * NR

_PREP_BN = 400


def _prep_body(x_ref, w_ref, a2_ref, h_ref, ab_ref):
    h = jnp.dot(x_ref[...], w_ref[...], preferred_element_type=jnp.float32)
    h_ref[...] = h
    ab_ref[...] = jnp.dot(h, a2_ref[...], preferred_element_type=jnp.float32)


def _prep(x, W, A2):
    return pl.pallas_call(
        _prep_body,
        grid=(N // _PREP_BN,),
        in_specs=[
            pl.BlockSpec((_PREP_BN, D), lambda i: (i, 0)),
            pl.BlockSpec((D, D), lambda i: (0, 0)),
            pl.BlockSpec((D, 2), lambda i: (0, 0)),
        ],
        out_specs=[
            pl.BlockSpec((_PREP_BN, D), lambda i: (i, 0)),
            pl.BlockSpec((_PREP_BN, 2), lambda i: (i, 0)),
        ],
        out_shape=[
            jax.ShapeDtypeStruct((N, D), jnp.float32),
            jax.ShapeDtypeStruct((N, 2), jnp.float32),
        ],
    )(x, W, A2)


_sc_mesh = plsc.VectorSubcoreMesh(core_axis_name="c", subcore_axis_name="s")


@functools.partial(
    pl.kernel,
    out_type=(
        jax.ShapeDtypeStruct((NC, ACC_ROWS, D), jnp.float32),
        jax.ShapeDtypeStruct((NC, DC_ROWS, D), jnp.float32),
    ),
    mesh=_sc_mesh,
    compiler_params=pltpu.CompilerParams(needs_layout_passes=False),
    scratch_types=[
        pltpu.VMEM((N_PAD,), jnp.float32),
        pltpu.VMEM((N_PAD,), jnp.float32),
        pltpu.VMEM((CPT, CH), jnp.int32),
        pltpu.VMEM((CPT, CH), jnp.int32),
        pltpu.VMEM((CPT, CH), jnp.int32),
        pltpu.VMEM((CPT * CH, D), jnp.float32),
        pltpu.VMEM((NR, D), jnp.float32),
        pltpu.VMEM((NR, D), jnp.float32),
        pltpu.VMEM((2, NR), jnp.int32),
        pltpu.VMEM_SHARED((ACC_ROWS, D), jnp.float32),
        pltpu.VMEM_SHARED((DC_ROWS, D), jnp.float32),
        pltpu.SemaphoreType.DMA,
    ],
)
def _sc_edges(as_hbm, ar_hbm, sidx_hbm, ridx_hbm, h_hbm, zrows_hbm, iota_hbm,
              msg_hbm, dc_hbm,
              as_v, ar_v, sidx_v, ridx_v, lidx_v, gbuf, den_v, cnt_v, iota_v,
              acc, dc_acc, sem):
    cid = lax.axis_index("c")
    sid = lax.axis_index("s")
    half_lo = cid * HALF

    pltpu.sync_copy(as_hbm, as_v)
    pltpu.sync_copy(ar_hbm, ar_v)
    pltpu.sync_copy(iota_hbm, iota_v)
    pltpu.sync_copy(zrows_hbm.at[pl.ds(0, NR)], den_v)
    pltpu.sync_copy(zrows_hbm.at[pl.ds(0, NR)], cnt_v)
    pltpu.sync_copy(zrows_hbm.at[pl.ds(0, ACC_PER_TILE)],
                    acc.at[pl.ds(sid * ACC_PER_TILE, ACC_PER_TILE)])

    @pl.when(sid < DC_ROWS // 16)
    def _zero_dc():
        pltpu.sync_copy(zrows_hbm.at[pl.ds(0, 16)],
                        dc_acc.at[pl.ds(sid * 16, 16)])

    plsc.subcore_barrier()

    row0 = sid * (EPT // CH)

    def trip_body(t, carry):
        pltpu.sync_copy(sidx_hbm.at[pl.ds(row0 + t * CPT, CPT)], sidx_v)
        pltpu.sync_copy(ridx_hbm.at[pl.ds(row0 + t * CPT, CPT)], ridx_v)
        for k in range(CPT):
            pltpu.async_copy(h_hbm.at[sidx_v.at[k]],
                             gbuf.at[pl.ds(k * CH, CH)], sem).wait()
            for g in range(CH // L):
                s16 = sidx_v[k, pl.ds(g * L, L)]
                r16 = ridx_v[k, pl.ds(g * L, L)]
                sc = plsc.load_gather(as_v, [s16]) + plsc.load_gather(ar_v, [r16])
                sc = jnp.where(sc >= 0.0, sc, 0.01 * sc)
                w16 = jnp.exp(sc)
                l16 = r16 - half_lo
                valid = jnp.logical_and(l16 >= 0, l16 < HALF)
                lidx_v[k, pl.ds(g * L, L)] = jnp.where(valid, l16, JUNK)
                rdiv = lax.shift_right_logical(r16, 7)
                rmod = lax.bitwise_and(r16, 127)
                zero = jnp.zeros((L,), jnp.float32)
                one = jnp.ones((L,), jnp.float32)
                plsc.addupdate_scatter(den_v, [rdiv, rmod],
                                       jnp.where(valid, w16, zero))
                plsc.addupdate_scatter(cnt_v, [rdiv, rmod],
                                       jnp.where(valid, one, zero))
                for i in range(L):
                    e = k * CH + g * L + i
                    w = jnp.full((L,), w16[i])
                    for v in range(D // L):
                        gbuf[e, pl.ds(v * L, L)] = gbuf[e, pl.ds(v * L, L)] * w
            pltpu.sync_copy(gbuf.at[pl.ds(k * CH, CH)], acc.at[lidx_v.at[k]],
                            add=True)
        return carry

    lax.fori_loop(0, NTRIP, trip_body, 0)
    pltpu.sync_copy(den_v, dc_acc.at[iota_v.at[0]], add=True)
    pltpu.sync_copy(cnt_v, dc_acc.at[iota_v.at[1]], add=True)
    plsc.subcore_barrier()
    pltpu.sync_copy(acc.at[pl.ds(sid * ACC_PER_TILE, ACC_PER_TILE)],
                    msg_hbm.at[cid, pl.ds(sid * ACC_PER_TILE, ACC_PER_TILE)])

    @pl.when(sid < DC_ROWS // 16)
    def _write_dc():
        pltpu.sync_copy(dc_acc.at[pl.ds(sid * 16, 16)],
                        dc_hbm.at[cid, pl.ds(sid * 16, 16)])


_FIN_BN = 400


def _fin_body(m_ref, d_ref, c_ref, o_ref):
    den = d_ref[0] + d_ref[1]
    cnt = c_ref[0] + c_ref[1]
    o_ref[...] = m_ref[...] / (jnp.maximum(den, 1e-30) * jnp.maximum(cnt, 1.0))


def _finalize(msg, den, cnt):
    return pl.pallas_call(
        _fin_body,
        grid=(N // _FIN_BN,),
        in_specs=[
            pl.BlockSpec((_FIN_BN, D), lambda i: (i, 0)),
            pl.BlockSpec((NC, _FIN_BN, 1), lambda i: (0, i, 0)),
            pl.BlockSpec((NC, _FIN_BN, 1), lambda i: (0, i, 0)),
        ],
        out_specs=pl.BlockSpec((_FIN_BN, D), lambda i: (i, 0)),
        out_shape=jax.ShapeDtypeStruct((N, D), jnp.float32),
    )(msg, den, cnt)


def kernel(x, edge_index, W, A):
    A2 = jnp.stack([A[:D, 0], A[D:, 0]], axis=1)
    h, ab = _prep(x, W, A2)
    a_s = jnp.pad(ab[:, 0], (0, N_PAD - N))
    a_r = jnp.pad(ab[:, 1], (0, N_PAD - N))
    pad = E_PAD - E
    senders = jnp.concatenate(
        [edge_index[0], jnp.zeros((pad,), jnp.int32)]).reshape(E_PAD // CH, CH)
    receivers = jnp.concatenate(
        [edge_index[1],
         jnp.full((pad,), PAD_RECV, jnp.int32)]).reshape(E_PAD // CH, CH)
    zrows = jnp.zeros((ACC_PER_TILE, D), jnp.float32)
    iotas = jnp.arange(DC_ROWS, dtype=jnp.int32).reshape(2, NR)
    msg, dc = _sc_edges(a_s, a_r, senders, receivers, h, zrows, iotas)
    msg_full = jnp.concatenate([msg[0, :HALF], msg[1, :HALF]], axis=0)
    dc_flat = dc.reshape(NC, 2 * N_PAD)
    den = dc_flat[:, :N_PAD].reshape(NC, N_PAD, 1)
    cnt = dc_flat[:, N_PAD:].reshape(NC, N_PAD, 1)
    return _finalize(msg_full, den, cnt)


# speedup vs baseline: 8.1694x; 1.1342x over previous
import functools

import jax
import jax.numpy as jnp
from jax import lax
from jax.experimental import pallas as pl
from jax.experimental.pallas import tpu as pltpu
from jax.experimental.pallas import tpu_sc as plsc

N = 10000
E = 320000
D = 128

NC = 2
NS = 16
L = 16
NW = NC * NS

N_PAD = 10016
HALF = N_PAD // NC
ACC_ROWS = 5120
ACC_PER_TILE = ACC_ROWS // NS
JUNK = HALF + 32
HR = 40
CH = 128
NBUF = 2
E_PAD = 327680
EPT = E_PAD // NS
NCHUNK = EPT // CH
NTRIP = NCHUNK // NBUF
PAD_RECV = 10008
DC_ROWS = 2 * HR

_PREP_BN = 400


def _prep_body(x_ref, w_ref, a2_ref, h_ref, ab_ref):
    h = jnp.dot(x_ref[...], w_ref[...], preferred_element_type=jnp.float32)
    h_ref[...] = h
    ab_ref[...] = jnp.dot(h, a2_ref[...], preferred_element_type=jnp.float32)


def _prep(x, W, A2):
    return pl.pallas_call(
        _prep_body,
        grid=(N // _PREP_BN,),
        in_specs=[
            pl.BlockSpec((_PREP_BN, D), lambda i: (i, 0)),
            pl.BlockSpec((D, D), lambda i: (0, 0)),
            pl.BlockSpec((D, 2), lambda i: (0, 0)),
        ],
        out_specs=[
            pl.BlockSpec((_PREP_BN, D), lambda i: (i, 0)),
            pl.BlockSpec((_PREP_BN, 2), lambda i: (i, 0)),
        ],
        out_shape=[
            jax.ShapeDtypeStruct((N, D), jnp.float32),
            jax.ShapeDtypeStruct((N, 2), jnp.float32),
        ],
    )(x, W, A2)


_sc_mesh = plsc.VectorSubcoreMesh(core_axis_name="c", subcore_axis_name="s")


@functools.partial(
    pl.kernel,
    out_type=(
        jax.ShapeDtypeStruct((NC, ACC_ROWS, D), jnp.float32),
        jax.ShapeDtypeStruct((NC, DC_ROWS, D), jnp.float32),
    ),
    mesh=_sc_mesh,
    compiler_params=pltpu.CompilerParams(needs_layout_passes=False),
    scratch_types=[
        pltpu.VMEM((N_PAD,), jnp.float32),
        pltpu.VMEM((N_PAD,), jnp.float32),
        pltpu.VMEM((NBUF, 2, CH), jnp.int32),
        pltpu.VMEM((NBUF, CH), jnp.int32),
        pltpu.VMEM((NBUF, CH, D), jnp.float32),
        pltpu.VMEM((HR, D), jnp.float32),
        pltpu.VMEM((HR, D), jnp.float32),
        pltpu.VMEM((2, HR), jnp.int32),
        pltpu.VMEM_SHARED((ACC_ROWS, D), jnp.float32),
        pltpu.VMEM_SHARED((DC_ROWS, D), jnp.float32),
        pltpu.SemaphoreType.DMA((NBUF,)),
        pltpu.SemaphoreType.DMA((NBUF,)),
        pltpu.SemaphoreType.DMA((NBUF,)),
    ],
)
def _sc_edges(as_hbm, ar_hbm, sridx_hbm, h_hbm, zrows_hbm, iota_hbm,
              msg_hbm, dc_hbm,
              as_v, ar_v, sridx_v, lidx_v, gbuf, den_v, cnt_v, iota_v,
              acc, dc_acc, sem_g, sem_s, sem_i):
    cid = lax.axis_index("c")
    sid = lax.axis_index("s")
    half_lo = cid * HALF

    pltpu.sync_copy(as_hbm, as_v)
    pltpu.sync_copy(ar_hbm, ar_v)
    pltpu.sync_copy(iota_hbm, iota_v)
    pltpu.sync_copy(zrows_hbm.at[pl.ds(0, HR)], den_v)
    pltpu.sync_copy(zrows_hbm.at[pl.ds(0, HR)], cnt_v)
    pltpu.sync_copy(zrows_hbm, gbuf.at[0])
    for _q in range(ACC_PER_TILE // CH):
        pltpu.sync_copy(gbuf.at[0],
                        acc.at[pl.ds(sid * ACC_PER_TILE + _q * CH, CH)])
    _rem = ACC_PER_TILE % CH
    if _rem:
        pltpu.sync_copy(
            gbuf.at[0, pl.ds(0, _rem)],
            acc.at[pl.ds(sid * ACC_PER_TILE + (ACC_PER_TILE // CH) * CH,
                         _rem)])

    @pl.when(sid < DC_ROWS // 16)
    def _zero_dc():
        pltpu.sync_copy(gbuf.at[0, pl.ds(0, 16)],
                        dc_acc.at[pl.ds(sid * 16, 16)])

    plsc.subcore_barrier()

    row0 = sid * NCHUNK
    junk16 = jnp.full((L,), JUNK, jnp.int32)

    pltpu.sync_copy(sridx_hbm.at[pl.ds(row0, 1)], sridx_v.at[pl.ds(0, 1)])
    pltpu.async_copy(h_hbm.at[sridx_v.at[0, 0]], gbuf.at[0], sem_g.at[0])
    pltpu.async_copy(sridx_hbm.at[pl.ds(row0 + 1, 1)],
                     sridx_v.at[pl.ds(1, 1)], sem_i.at[1])

    def trip_body(t, carry):
        for k in range(NBUF):
            c = t * NBUF + k
            k1 = (k + 1) % NBUF
            pltpu.make_async_copy(h_hbm.at[sridx_v.at[k, 0]], gbuf.at[k],
                                  sem_g.at[k]).wait()

            @pl.when(c >= NBUF - 1)
            def _drain_prev():
                pltpu.make_async_copy(gbuf.at[k1], acc.at[lidx_v.at[k1]],
                                      sem_s.at[k1]).wait()

            @pl.when(c + 1 < NCHUNK)
            def _issue_gather():
                pltpu.make_async_copy(sridx_hbm.at[pl.ds(row0 + c + 1, 1)],
                                      sridx_v.at[pl.ds(k1, 1)],
                                      sem_i.at[k1]).wait()
                pltpu.async_copy(h_hbm.at[sridx_v.at[k1, 0]], gbuf.at[k1],
                                 sem_g.at[k1])

            for g in range(CH // L):
                s16 = sridx_v[k, 0, pl.ds(g * L, L)]
                r16 = sridx_v[k, 1, pl.ds(g * L, L)]
                sc = plsc.load_gather(as_v, [s16]) + plsc.load_gather(ar_v, [r16])
                sc = jnp.where(sc >= 0.0, sc, 0.01 * sc)
                w16 = jnp.exp(sc)
                l16 = r16 - half_lo
                valid = jnp.logical_and(l16 >= 0, l16 < HALF)
                lidx_v[k, pl.ds(g * L, L)] = jnp.where(valid, l16, junk16)
                lc = jnp.where(valid, l16, 0)
                ldiv = lax.shift_right_logical(lc, 7)
                lmod = lax.bitwise_and(lc, 127)
                zero = jnp.zeros((L,), jnp.float32)
                one = jnp.ones((L,), jnp.float32)
                plsc.addupdate_scatter(den_v, [ldiv, lmod],
                                       jnp.where(valid, w16, zero))
                plsc.addupdate_scatter(cnt_v, [ldiv, lmod],
                                       jnp.where(valid, one, zero))
                for i in range(L):
                    e = g * L + i
                    w = jnp.full((L,), w16[i])
                    for v in range(D // L):
                        gbuf[k, e, pl.ds(v * L, L)] = \
                            gbuf[k, e, pl.ds(v * L, L)] * w

            @pl.when(c + 2 < NCHUNK)
            def _stage_idx():
                pltpu.async_copy(sridx_hbm.at[pl.ds(row0 + c + 2, 1)],
                                 sridx_v.at[pl.ds(k, 1)], sem_i.at[k])

            pltpu.async_copy(gbuf.at[k], acc.at[lidx_v.at[k]], sem_s.at[k],
                             add=True)
        return carry

    lax.fori_loop(0, NTRIP, trip_body, 0)
    for j in range(NBUF - 1):
        kk = (NCHUNK - 1 - j) % NBUF
        pltpu.make_async_copy(gbuf.at[kk], acc.at[lidx_v.at[kk]],
                              sem_s.at[kk]).wait()
    pltpu.sync_copy(den_v, dc_acc.at[iota_v.at[0]], add=True)
    pltpu.sync_copy(cnt_v, dc_acc.at[iota_v.at[1]], add=True)
    plsc.subcore_barrier()
    pltpu.sync_copy(acc.at[pl.ds(sid * ACC_PER_TILE, ACC_PER_TILE)],
                    msg_hbm.at[cid, pl.ds(sid * ACC_PER_TILE, ACC_PER_TILE)])

    @pl.when(sid < DC_ROWS // 16)
    def _write_dc():
        pltpu.sync_copy(dc_acc.at[pl.ds(sid * 16, 16)],
                        dc_hbm.at[cid, pl.ds(sid * 16, 16)])


_FIN_BN = 400


def _fin_body(m_ref, d_ref, c_ref, o_ref):
    den = d_ref[...]
    cnt = c_ref[...]
    o_ref[...] = m_ref[...] / (jnp.maximum(den, 1e-30) * jnp.maximum(cnt, 1.0))


def _finalize(msg, den, cnt):
    return pl.pallas_call(
        _fin_body,
        grid=(N // _FIN_BN,),
        in_specs=[
            pl.BlockSpec((_FIN_BN, D), lambda i: (i, 0)),
            pl.BlockSpec((_FIN_BN, 1), lambda i: (i, 0)),
            pl.BlockSpec((_FIN_BN, 1), lambda i: (i, 0)),
        ],
        out_specs=pl.BlockSpec((_FIN_BN, D), lambda i: (i, 0)),
        out_shape=jax.ShapeDtypeStruct((N, D), jnp.float32),
    )(msg, den, cnt)


def kernel(x, edge_index, W, A):
    A2 = jnp.stack([A[:D, 0], A[D:, 0]], axis=1)
    h, ab = _prep(x, W, A2)
    a_s = jnp.pad(ab[:, 0], (0, N_PAD - N))
    a_r = jnp.pad(ab[:, 1], (0, N_PAD - N))
    pad = E_PAD - E
    senders = jnp.concatenate(
        [edge_index[0], jnp.zeros((pad,), jnp.int32)]).reshape(E_PAD // CH, CH)
    receivers = jnp.concatenate(
        [edge_index[1],
         jnp.full((pad,), PAD_RECV, jnp.int32)]).reshape(E_PAD // CH, CH)
    sridx = jnp.stack([senders, receivers], axis=1)
    zrows = jnp.zeros((CH, D), jnp.float32)
    iotas = jnp.arange(DC_ROWS, dtype=jnp.int32).reshape(2, HR)
    msg, dc = _sc_edges(a_s, a_r, sridx, h, zrows, iotas)
    msg_full = jnp.concatenate([msg[0, :HALF], msg[1, :HALF]], axis=0)
    den = jnp.concatenate([dc[0, :HR].reshape(HR * D)[:HALF],
                           dc[1, :HR].reshape(HR * D)[:HALF]]).reshape(N_PAD, 1)
    cnt = jnp.concatenate([dc[0, HR:].reshape(HR * D)[:HALF],
                           dc[1, HR:].reshape(HR * D)[:HALF]]).reshape(N_PAD, 1)
    return _finalize(msg_full, den, cnt)
